# baseline (speedup 1.0000x reference)
import functools

import jax
import jax.numpy as jnp
from jax import lax
from jax.experimental import pallas as pl
from jax.experimental.pallas import tpu as pltpu
from jax.experimental.pallas import tpu_sc as plsc

D = 128
NW_WALKS = 65536
CTX = 10

NC = 2
NS = 16
NWORK = NC * NS

CHUNK = 8
IDS_HALF = CHUNK * CTX
IDS_PER_CHUNK = 2 * IDS_HALF
PAIRS_PER_WORKER = NW_WALKS // NWORK
CHUNKS_PER_WORKER = PAIRS_PER_WORKER // CHUNK
GSLICE = 80
NSLICE = IDS_PER_CHUNK // GSLICE


def _make_sc_kernel():
  mesh = plsc.VectorSubcoreMesh(core_axis_name="c", subcore_axis_name="s")

  @functools.partial(
      pl.kernel,
      mesh=mesh,
      out_type=jax.ShapeDtypeStruct((NWORK, 16), jnp.float32),
      scratch_types=[
          pltpu.VMEM((IDS_PER_CHUNK,), jnp.int32),
          pltpu.VMEM((IDS_PER_CHUNK,), jnp.int32),
          pltpu.VMEM((IDS_PER_CHUNK,), jnp.int32),
          pltpu.VMEM((IDS_PER_CHUNK,), jnp.int32),
          pltpu.VMEM((IDS_PER_CHUNK,), jnp.int32),
          pltpu.VMEM((IDS_PER_CHUNK, D), jnp.float32),
          pltpu.VMEM((IDS_PER_CHUNK, D), jnp.float32),
          pltpu.VMEM((IDS_PER_CHUNK, D), jnp.float32),
          pltpu.VMEM((16,), jnp.float32),
          pltpu.VMEM_SHARED((100000,), jnp.int32),
          pltpu.SemaphoreType.DMA,
          pltpu.SemaphoreType.DMA,
          pltpu.SemaphoreType.DMA,
          pltpu.SemaphoreType.DMA,
          pltpu.SemaphoreType.DMA,
          pltpu.SemaphoreType.DMA,
          pltpu.SemaphoreType.DMA,
          pltpu.SemaphoreType.DMA,
      ],
      compiler_params=pltpu.CompilerParams(needs_layout_passes=False),
  )
  def sc_kernel(pos_hbm, neg_hbm, map_hbm, emb_hbm, out_hbm,
                ids0_v, ids1_v, mid0_v, mid1_v, mid2_v,
                rows0_v, rows1_v, rows2_v, outv,
                map_sh, si0, si1, sm0, sm1, sm2, sr0, sr1, sr2):
    wid = lax.axis_index("s") * NC + lax.axis_index("c")
    lane = lax.iota(jnp.int32, 16)
    ids_b = (ids0_v, ids1_v)
    mid_b = (mid0_v, mid1_v, mid2_v)
    rows = (rows0_v, rows1_v, rows2_v)
    sem_i = (si0, si1)
    sem_m = (sm0, sm1, sm2)
    sem_r = (sr0, sr1, sr2)
    base = wid * (PAIRS_PER_WORKER * CTX)

    def ids_copies(c, p, make):
      off = pl.multiple_of(
          base + lax.rem(c, CHUNKS_PER_WORKER) * IDS_HALF, 8)
      return [
          make(pos_hbm.at[pl.ds(off, IDS_HALF)],
               ids_b[p].at[pl.ds(0, IDS_HALF)], sem_i[p]),
          make(neg_hbm.at[pl.ds(off, IDS_HALF)],
               ids_b[p].at[pl.ds(IDS_HALF, IDS_HALF)], sem_i[p]),
      ]

    def map_copies(pi, pm, make):
      return [
          make(map_sh.at[ids_b[pi].at[pl.ds(k * GSLICE, GSLICE)]],
               mid_b[pm].at[pl.ds(k * GSLICE, GSLICE)], sem_m[pm])
          for k in range(NSLICE)
      ]

    def row_copies(pm, make):
      return [
          make(emb_hbm.at[mid_b[pm].at[pl.ds(k * GSLICE, GSLICE)]],
               rows[pm].at[pl.ds(k * GSLICE, GSLICE)], sem_r[pm])
          for k in range(NSLICE)
      ]

    issue = pltpu.async_copy

    def drain(cps):
      for cp in cps:
        cp.wait()

    def wait(make_list_fn, *args):
      for cp in make_list_fn(*args, pltpu.make_async_copy):
        cp.wait()

    K1 = D // 16

    def compute(rows_v, tot):
      for half in range((CHUNK + 15) // 16):
        wbase = half * 16
        nwalk = min(16, CHUNK - wbase)

        def walk_body(w, accs):
          rp = (wbase + w) * CTX
          rn = IDS_HALF + (wbase + w) * CTX
          hp = [rows_v[rp, pl.ds(k * 16, 16)] for k in range(K1)]
          hn = [rows_v[rn, pl.ds(k * 16, 16)] for k in range(K1)]
          out = []
          for j in range(1, CTX):
            t = hp[0] * rows_v[rp + j, pl.ds(0, 16)]
            for k in range(1, K1):
              t = t + hp[k] * rows_v[rp + j, pl.ds(k * 16, 16)]
            out.append(jnp.where(lane == w, jnp.sum(t), accs[j - 1]))
          for j in range(1, CTX):
            t = hn[0] * rows_v[rn + j, pl.ds(0, 16)]
            for k in range(1, K1):
              t = t + hn[k] * rows_v[rn + j, pl.ds(k * 16, 16)]
            out.append(jnp.where(lane == w, jnp.sum(t), accs[8 + j]))
          return tuple(out)

        zero = jnp.zeros((16,), jnp.float32)
        accs = lax.fori_loop(0, nwalk, walk_body,
                             tuple(zero for _ in range(18)))

        m = accs[0]
        for a in accs[1:]:
          m = jnp.maximum(m, a)
        sp = jnp.zeros((16,), jnp.float32)
        sn = jnp.zeros((16,), jnp.float32)
        for j in range(9):
          sp = sp + jnp.exp(accs[j] - m)
          sn = sn + jnp.exp(accs[9 + j] - m)
        r = sp / (sp + sn)
        if nwalk < 16:
          r = jnp.where(lane < nwalk, r, jnp.zeros((16,), jnp.float32))
        tot = tot + r
      return tot

    sid = lax.axis_index("s")
    @pl.when(sid == 0)
    def _():
      pltpu.sync_copy(map_hbm, map_sh)
    plsc.subcore_barrier()

    drain(ids_copies(0, 0, issue))
    drain(map_copies(0, 0, issue))
    drain(ids_copies(1, 1, issue))
    drain(map_copies(1, 1, issue))
    drain(ids_copies(2, 0, issue))
    row_copies(0, issue)
    row_copies(1, issue)
    map_copies(0, 2, issue)
    ids_copies(3, 1, issue)

    def step(c, p3, p2, tot):
      n3 = (p3 + 2) % 3
      wait(row_copies, p3)
      wait(map_copies, p2, n3)
      row_copies(n3, issue)
      ids_copies(c + 4, p2, issue)
      wait(ids_copies, c + 3, 1 - p2)
      map_copies(1 - p2, p3, issue)
      return compute(rows[p3], tot)

    def six_body(i, tot):
      c = 6 * i
      for k in range(6):
        tot = step(c + k, k % 3, k % 2, tot)
      return tot

    NFULL = (CHUNKS_PER_WORKER // 6) * 6
    tot = lax.fori_loop(0, CHUNKS_PER_WORKER // 6, six_body,
                        jnp.zeros((16,), jnp.float32))
    for c in range(NFULL, CHUNKS_PER_WORKER):
      tot = step(c, c % 3, c % 2, tot)

    wait(row_copies, CHUNKS_PER_WORKER % 3)
    wait(row_copies, (CHUNKS_PER_WORKER + 1) % 3)
    wait(map_copies, CHUNKS_PER_WORKER % 2, (CHUNKS_PER_WORKER + 2) % 3)
    wait(ids_copies, CHUNKS_PER_WORKER + 3, (CHUNKS_PER_WORKER + 1) % 2)

    outv[...] = tot
    pltpu.sync_copy(outv, out_hbm.at[wid])

  return sc_kernel


_SC_KERNEL = _make_sc_kernel()


def kernel(pos_rw, neg_rw, mapping, embedding):
  partials = _SC_KERNEL(
      pos_rw.reshape(-1).astype(jnp.int32),
      neg_rw.reshape(-1).astype(jnp.int32),
      mapping.astype(jnp.int32),
      embedding.astype(jnp.float32))
  return -(jnp.sum(partials) / jnp.float32(NW_WALKS))


# speedup vs baseline: 1.3631x; 1.3631x over previous
import functools

import jax
import jax.numpy as jnp
from jax import lax
from jax.experimental import pallas as pl
from jax.experimental.pallas import tpu as pltpu
from jax.experimental.pallas import tpu_sc as plsc

D = 128
NW_WALKS = 65536
CTX = 10

NC = 2
NS = 16
NWORK = NC * NS

CHUNK = 16
IDS_HALF = CHUNK * CTX
IDS_PER_CHUNK = 2 * IDS_HALF
PAIRS_PER_WORKER = NW_WALKS // NWORK
CHUNKS_PER_WORKER = PAIRS_PER_WORKER // CHUNK
GSLICE = 80
NSLICE = IDS_PER_CHUNK // GSLICE


def _make_sc_kernel():
  mesh = plsc.VectorSubcoreMesh(core_axis_name="c", subcore_axis_name="s")

  @functools.partial(
      pl.kernel,
      mesh=mesh,
      out_type=jax.ShapeDtypeStruct((NWORK, 16), jnp.float32),
      scratch_types=[
          pltpu.VMEM((IDS_PER_CHUNK,), jnp.int32),
          pltpu.VMEM((IDS_PER_CHUNK,), jnp.int32),
          pltpu.VMEM((IDS_PER_CHUNK,), jnp.int32),
          pltpu.VMEM((IDS_PER_CHUNK,), jnp.int32),
          pltpu.VMEM((IDS_PER_CHUNK, D), jnp.float32),
          pltpu.VMEM((IDS_PER_CHUNK, D), jnp.float32),
          pltpu.VMEM((16,), jnp.float32),
          pltpu.VMEM_SHARED((100000,), jnp.int32),
          pltpu.SemaphoreType.DMA,
          pltpu.SemaphoreType.DMA,
          pltpu.SemaphoreType.DMA,
          pltpu.SemaphoreType.DMA,
          pltpu.SemaphoreType.DMA,
          pltpu.SemaphoreType.DMA,
      ],
      compiler_params=pltpu.CompilerParams(needs_layout_passes=False),
  )
  def sc_kernel(pos_hbm, neg_hbm, map_hbm, emb_hbm, out_hbm,
                ids0_v, ids1_v, mid0_v, mid1_v, rows0_v, rows1_v, outv,
                map_sh, si0, si1, sm0, sm1, sr0, sr1):
    wid = lax.axis_index("s") * NC + lax.axis_index("c")
    lane = lax.iota(jnp.int32, 16)
    ids_b = (ids0_v, ids1_v)
    mid_b = (mid0_v, mid1_v)
    rows = (rows0_v, rows1_v)
    sem_i = (si0, si1)
    sem_m = (sm0, sm1)
    sem_r = (sr0, sr1)
    base = wid * (PAIRS_PER_WORKER * CTX)

    def ids_copies(c, p, make):
      off = pl.multiple_of(
          base + lax.rem(c, CHUNKS_PER_WORKER) * IDS_HALF, 8)
      return [
          make(pos_hbm.at[pl.ds(off, IDS_HALF)],
               ids_b[p].at[pl.ds(0, IDS_HALF)], sem_i[p]),
          make(neg_hbm.at[pl.ds(off, IDS_HALF)],
               ids_b[p].at[pl.ds(IDS_HALF, IDS_HALF)], sem_i[p]),
      ]

    def map_copies(p, make):
      return [
          make(map_sh.at[ids_b[p].at[pl.ds(k * GSLICE, GSLICE)]],
               mid_b[p].at[pl.ds(k * GSLICE, GSLICE)], sem_m[p])
          for k in range(NSLICE)
      ]

    def row_copies(p, make):
      return [
          make(emb_hbm.at[mid_b[p].at[pl.ds(k * GSLICE, GSLICE)]],
               rows[p].at[pl.ds(k * GSLICE, GSLICE)], sem_r[p])
          for k in range(NSLICE)
      ]

    issue = pltpu.async_copy

    def drain(cps):
      for cp in cps:
        cp.wait()

    def wait(make_list_fn, *args):
      for cp in make_list_fn(*args, pltpu.make_async_copy):
        cp.wait()

    K1 = D // 16

    def compute(rows_v, tot):
      for half in range(CHUNK // 16):
        wbase = half * 16

        def walk_body(w, accs):
          rp = (wbase + w) * CTX
          rn = IDS_HALF + (wbase + w) * CTX
          hp = [rows_v[rp, pl.ds(k * 16, 16)] for k in range(K1)]
          hn = [rows_v[rn, pl.ds(k * 16, 16)] for k in range(K1)]
          out = []
          for j in range(1, CTX):
            t = hp[0] * rows_v[rp + j, pl.ds(0, 16)]
            for k in range(1, K1):
              t = t + hp[k] * rows_v[rp + j, pl.ds(k * 16, 16)]
            out.append(jnp.where(lane == w, jnp.sum(t), accs[j - 1]))
          for j in range(1, CTX):
            t = hn[0] * rows_v[rn + j, pl.ds(0, 16)]
            for k in range(1, K1):
              t = t + hn[k] * rows_v[rn + j, pl.ds(k * 16, 16)]
            out.append(jnp.where(lane == w, jnp.sum(t), accs[8 + j]))
          return tuple(out)

        zero = jnp.zeros((16,), jnp.float32)
        accs = lax.fori_loop(0, 16, walk_body,
                             tuple(zero for _ in range(18)))

        m = accs[0]
        for a in accs[1:]:
          m = jnp.maximum(m, a)
        sp = jnp.zeros((16,), jnp.float32)
        sn = jnp.zeros((16,), jnp.float32)
        for j in range(9):
          sp = sp + jnp.exp(accs[j] - m)
          sn = sn + jnp.exp(accs[9 + j] - m)
        tot = tot + sp / (sp + sn)
      return tot

    sid = lax.axis_index("s")
    @pl.when(sid == 0)
    def _():
      pltpu.sync_copy(map_hbm, map_sh)
    plsc.subcore_barrier()

    drain(ids_copies(0, 0, issue))
    drain(map_copies(0, issue))
    drain(ids_copies(1, 1, issue))
    row_copies(0, issue)
    map_copies(1, issue)
    ids_copies(2, 0, issue)

    def step(c, p, tot):
      q = 1 - p
      wait(map_copies, q)
      row_copies(q, issue)
      wait(row_copies, p)
      ids_copies(c + 3, q, issue)
      wait(ids_copies, c + 2, p)
      map_copies(p, issue)
      return compute(rows[p], tot)

    def pair_body(i, tot):
      tot = step(2 * i, 0, tot)
      tot = step(2 * i + 1, 1, tot)
      return tot

    tot = lax.fori_loop(0, CHUNKS_PER_WORKER // 2, pair_body,
                        jnp.zeros((16,), jnp.float32))

    wait(row_copies, 0)
    wait(map_copies, 1)
    wait(ids_copies, 2, 0)

    outv[...] = tot
    pltpu.sync_copy(outv, out_hbm.at[wid])

  return sc_kernel


_SC_KERNEL = _make_sc_kernel()


def kernel(pos_rw, neg_rw, mapping, embedding):
  partials = _SC_KERNEL(
      pos_rw.reshape(-1).astype(jnp.int32),
      neg_rw.reshape(-1).astype(jnp.int32),
      mapping.astype(jnp.int32),
      embedding.astype(jnp.float32))
  return -(jnp.sum(partials) / jnp.float32(NW_WALKS))
